# baseline (speedup 1.0000x reference)
import functools
import jax
import jax.numpy as jnp
from jax import lax
from jax.experimental import pallas as pl
from jax.experimental.pallas import tpu as pltpu
from jax.experimental.pallas import tpu_sc as plsc

P = 1000
K = 15
L = 7
KS = 5
FNR = 10
EPS = 1e-8
PBLK = 200
PB = P // PBLK
BIG = 1e30
SENTF = 1.7014118346046923e38
LM = L * KS
PAD = 1008
NG = PAD // 16

_IP = False


def _k1(posPr_ref, pos3c_ref, dirs_ref, idx_ref, maxd2_ref, proj_ref):
    b = pl.program_id(0)
    rb = pl.program_id(1)
    d2 = jnp.zeros((PBLK, P), jnp.float32)
    for c in range(3):
        pr = posPr_ref[0, :, c:c + 1]
        pc = pos3c_ref[0, c:c + 1, :]
        diff = pc - pr
        d2 = d2 + diff * diff
    riota = lax.broadcasted_iota(jnp.int32, (PBLK, P), 0) + rb * PBLK
    ciota = lax.broadcasted_iota(jnp.int32, (PBLK, P), 1)
    d2 = jnp.where(riota == ciota, BIG, d2)
    keys = (lax.bitcast_convert_type(d2, jnp.int32) & (-1024)) | ciota
    kf = lax.bitcast_convert_type(keys + 0x00800000, jnp.float32)
    mkf = jnp.min(kf, axis=1, keepdims=True)
    idx_cols = []
    for i in range(K):
        if i > 0:
            mkf = jnp.min(jnp.where(kf > mkf, kf, SENTF), axis=1,
                          keepdims=True)
        mki = lax.bitcast_convert_type(mkf, jnp.int32) - 0x00800000
        idx_cols.append(mki & 1023)
    idx_ref[0, :, :] = jnp.concatenate(idx_cols, axis=1)
    sel = kf <= mkf
    smax = jnp.max(jnp.where(sel, d2, -1.0), axis=(0, 1), keepdims=True)

    @pl.when((b == 0) & (rb == 0))
    def _():
        maxd2_ref[:, :] = jnp.full((1, 1), -1.0, jnp.float32)

    maxd2_ref[:, :] = jnp.maximum(maxd2_ref[:, :], smax)

    @pl.when(rb == 0)
    def _():
        dirs = dirs_ref[...]
        n = jnp.sqrt(jnp.sum(dirs * dirs, axis=1, keepdims=True))
        du = dirs / (n + EPS)
        proj_ref[0, :, :] = jnp.dot(du, pos3c_ref[0],
                                    preferred_element_type=jnp.float32)


def _sc_body(pos_hbm, proj_hbm, idx_hbm, scale_hbm, out_hbm,
             posv, projv, idxv, scalev, abuf):
    wid = lax.axis_index("s") * 2 + lax.axis_index("c")
    pltpu.sync_copy(pos_hbm.at[wid], posv)
    pltpu.sync_copy(proj_hbm.at[wid], projv)
    pltpu.sync_copy(idx_hbm.at[wid], idxv)
    pltpu.sync_copy(scale_hbm, scalev)
    scale = scalev[...]
    lane = lax.broadcasted_iota(jnp.int32, (16,), 0)

    def group(g, carry):
        i0 = g * 16
        nd = jnp.minimum(i0 + lane, P - 1)
        pd = [plsc.load_gather(posv, [nd + c * P]) for c in range(3)]
        prd = [plsc.load_gather(projv, [nd + l * P]) for l in range(L)]
        acc = [jnp.zeros((16,), jnp.float32) for _ in range(LM)]
        iofs = nd * K
        for k in range(K):
            nidx = plsc.load_gather(idxv, [iofs + k])
            d2 = jnp.zeros((16,), jnp.float32)
            for c in range(3):
                ps = plsc.load_gather(posv, [nidx + c * P])
                df = ps - pd[c]
                d2 = d2 + df * df
            x = d2 + EPS
            xb = lax.bitcast_convert_type(x, jnp.int32)
            y = lax.bitcast_convert_type(
                (xb >> 1) + 0x1FBD1DF5, jnp.float32)
            for _ in range(3):
                y = 0.5 * (y + x / y)
            dist = y
            invd = 1.0 / (dist + EPS)
            t = dist * scale
            basis = [jnp.maximum(1.0 - jnp.abs(t - float(m)), 0.0)
                     for m in range(KS)]
            for l in range(L):
                pj = plsc.load_gather(projv, [nidx + l * P])
                dw = jnp.maximum((pj - prd[l]) * invd, 0.0)
                for m in range(KS):
                    acc[l * KS + m] = acc[l * KS + m] + dw * basis[m]
        ofs = (i0 + lane) * LM
        for lm in range(LM):
            plsc.store_scatter(abuf, [ofs + lm], acc[lm])
        return carry

    lax.fori_loop(0, NG, group, 0)
    pltpu.sync_copy(abuf, out_hbm.at[wid])


def _k3(a_ref, wf_ref, bdsc_ref, w1_ref, b1_ref, w2_ref, b2_ref,
        out_ref, ys_scr):
    b = pl.program_id(0)
    nb = pl.num_programs(0)
    a = a_ref[0]
    msg = jnp.dot(a, wf_ref[...], preferred_element_type=jnp.float32) / K
    msg = msg + bdsc_ref[...]
    y = 1.0 / (1.0 + jnp.exp(-msg))
    valid = lax.broadcasted_iota(jnp.int32, (PAD, FNR), 0) < P
    y = jnp.where(valid, y, 0.0)
    ys_scr[pl.ds(b, 1), :] = jnp.sum(y, axis=0, keepdims=True)

    @pl.when(b == nb - 1)
    def _():
        ys = ys_scr[...] * (1.0 / P)
        z = jnp.dot(ys, w1_ref[...],
                    preferred_element_type=jnp.float32) + b1_ref[...]
        h = jnp.where(z > 0, z, jnp.exp(jnp.minimum(z, 0.0)) - 1.0)
        logits = jnp.dot(h, w2_ref[...],
                         preferred_element_type=jnp.float32) + b2_ref[...]
        mx = jnp.max(logits, axis=1, keepdims=True)
        sh = logits - mx
        out_ref[...] = sh - jnp.log(jnp.sum(jnp.exp(sh), axis=1,
                                            keepdims=True))


def kernel(pos, edge_index, batch, dirs, W_spline, b_dsc, W1, b1, W2, b2):
    del edge_index, batch
    B = pos.shape[0] // P
    posB = pos.reshape(B, P, 3)
    pos3 = posB.transpose(0, 2, 1)
    wf = W_spline.reshape(LM, FNR)
    bdsc2 = b_dsc.reshape(1, FNR)

    idx, maxd2, proj = pl.pallas_call(
        _k1,
        grid=(B, PB),
        in_specs=[
            pl.BlockSpec((1, PBLK, 3), lambda b, rb: (b, rb, 0)),
            pl.BlockSpec((1, 3, P), lambda b, rb: (b, 0, 0)),
            pl.BlockSpec((L, 3), lambda b, rb: (0, 0)),
        ],
        out_specs=[
            pl.BlockSpec((1, PBLK, K), lambda b, rb: (b, rb, 0)),
            pl.BlockSpec((1, 1), lambda b, rb: (0, 0)),
            pl.BlockSpec((1, L, P), lambda b, rb: (b, 0, 0)),
        ],
        out_shape=[
            jax.ShapeDtypeStruct((B, P, K), jnp.int32),
            jax.ShapeDtypeStruct((1, 1), jnp.float32),
            jax.ShapeDtypeStruct((B, L, P), jnp.float32),
        ],
        interpret=_IP,
    )(posB, pos3, dirs)

    posp = pos3.reshape(B, 3 * P)
    projp = proj.reshape(B, L * P)
    idxp = idx.reshape(B, P * K)
    scale = (KS - 1) / (jnp.sqrt(maxd2[0, 0] + EPS) + EPS)
    scale16 = jnp.full((16,), 1.0, jnp.float32) * scale

    if _IP:
        a_t = _sc_emulate(posp, projp, idxp, scale16, B)
    else:
        mesh = plsc.VectorSubcoreMesh(core_axis_name="c", subcore_axis_name="s")
        sc = functools.partial(
            pl.kernel, _sc_body, mesh=mesh,
            compiler_params=pltpu.CompilerParams(needs_layout_passes=False),
            out_type=jax.ShapeDtypeStruct((B, PAD * LM), jnp.float32),
            scratch_types=[
                pltpu.VMEM((3 * P,), jnp.float32),
                pltpu.VMEM((L * P,), jnp.float32),
                pltpu.VMEM((P * K,), jnp.int32),
                pltpu.VMEM((16,), jnp.float32),
                pltpu.VMEM((PAD * LM,), jnp.float32),
            ],
        )()
        a_t = sc(posp, projp, idxp, scale16)
    a_t = a_t.reshape(B, PAD, LM)

    out = pl.pallas_call(
        _k3,
        grid=(B,),
        in_specs=[
            pl.BlockSpec((1, PAD, LM), lambda b: (b, 0, 0)),
            pl.BlockSpec((LM, FNR), lambda b: (0, 0)),
            pl.BlockSpec((1, FNR), lambda b: (0, 0)),
            pl.BlockSpec((FNR, 256), lambda b: (0, 0)),
            pl.BlockSpec((1, 256), lambda b: (0, 0)),
            pl.BlockSpec((256, W2.shape[1]), lambda b: (0, 0)),
            pl.BlockSpec((1, W2.shape[1]), lambda b: (0, 0)),
        ],
        out_specs=pl.BlockSpec((B, W2.shape[1]), lambda b: (0, 0)),
        out_shape=jax.ShapeDtypeStruct((B, W2.shape[1]), jnp.float32),
        scratch_shapes=[pltpu.VMEM((B, FNR), jnp.float32)],
        interpret=_IP,
    )(a_t, wf, bdsc2, W1, b1.reshape(1, -1), W2, b2.reshape(1, -1))
    return out


def _sc_emulate(posp, projp, idxp, scale16, B):
    pos = posp.reshape(B, 3, P)
    proj = projp.reshape(B, L, P)
    idx = idxp.reshape(B, P, K)
    scale = scale16[0]
    idxe = jnp.broadcast_to(idx.reshape(B, 1, P * K), (B, 3, P * K))
    g = jnp.take_along_axis(pos, idxe, axis=2).reshape(B, 3, P, K)
    d2 = jnp.sum((g - pos[:, :, :, None]) ** 2, axis=1)
    dist = jnp.sqrt(d2 + EPS)
    invd = 1.0 / (dist + EPS)
    t = dist * scale
    basis = jnp.maximum(1.0 - jnp.abs(t[..., None] -
                                      jnp.arange(KS, dtype=jnp.float32)), 0.0)
    idxe7 = jnp.broadcast_to(idx.reshape(B, 1, P * K), (B, L, P * K))
    pj = jnp.take_along_axis(proj, idxe7, axis=2).reshape(B, L, P, K)
    dw = jnp.maximum((pj - proj[:, :, :, None]) * invd[:, None], 0.0)
    a = jnp.einsum('blpk,bpkm->bplm', dw, basis).reshape(B, P, LM)
    a = jnp.pad(a, ((0, 0), (0, PAD - P), (0, 0)))
    return a.reshape(B, PAD * LM)


# speedup vs baseline: 1.0883x; 1.0883x over previous
import functools
import jax
import jax.numpy as jnp
from jax import lax
from jax.experimental import pallas as pl
from jax.experimental.pallas import tpu as pltpu
from jax.experimental.pallas import tpu_sc as plsc

P = 1000
K = 15
L = 7
KS = 5
FNR = 10
EPS = 1e-8
PBLK = 1000
PB = P // PBLK
BIG = 1e30
SENTF = 1.7014118346046923e38
LM = L * KS
PAD = 1008
NG = PAD // 16

_IP = False


def _k1(posPr_ref, pos3c_ref, dirs_ref, idx_ref, maxd2_ref, proj_ref):
    b = pl.program_id(0)
    rb = pl.program_id(1)
    d2 = jnp.zeros((PBLK, P), jnp.float32)
    for c in range(3):
        pr = posPr_ref[0, :, c:c + 1]
        pc = pos3c_ref[0, c:c + 1, :]
        diff = pc - pr
        d2 = d2 + diff * diff
    riota = lax.broadcasted_iota(jnp.int32, (PBLK, P), 0) + rb * PBLK
    ciota = lax.broadcasted_iota(jnp.int32, (PBLK, P), 1)
    d2 = jnp.where(riota == ciota, BIG, d2)
    keys = (lax.bitcast_convert_type(d2, jnp.int32) & (-1024)) | ciota
    kf = lax.bitcast_convert_type(keys + 0x00800000, jnp.float32)
    mkf = jnp.min(kf, axis=1, keepdims=True)
    idx_cols = []
    for i in range(K):
        if i > 0:
            mkf = jnp.min(jnp.where(kf > mkf, kf, SENTF), axis=1,
                          keepdims=True)
        mki = lax.bitcast_convert_type(mkf, jnp.int32) - 0x00800000
        idx_cols.append(mki & 1023)
    idx_ref[0, :, :] = jnp.concatenate(idx_cols, axis=1)
    sel = kf <= mkf
    smax = jnp.max(jnp.where(sel, d2, -1.0), axis=(0, 1), keepdims=True)

    @pl.when((b == 0) & (rb == 0))
    def _():
        maxd2_ref[:, :] = jnp.full((1, 1), -1.0, jnp.float32)

    maxd2_ref[:, :] = jnp.maximum(maxd2_ref[:, :], smax)

    @pl.when(rb == 0)
    def _():
        dirs = dirs_ref[...]
        n = jnp.sqrt(jnp.sum(dirs * dirs, axis=1, keepdims=True))
        du = dirs / (n + EPS)
        proj_ref[0, :, :] = jnp.dot(du, pos3c_ref[0],
                                    preferred_element_type=jnp.float32)


def _sc_body(pos_hbm, proj_hbm, idx_hbm, scale_hbm, out_hbm,
             posv, projv, idxv, scalev, abuf):
    wid = lax.axis_index("s") * 2 + lax.axis_index("c")
    pltpu.sync_copy(pos_hbm.at[wid], posv)
    pltpu.sync_copy(proj_hbm.at[wid], projv)
    pltpu.sync_copy(idx_hbm.at[wid], idxv)
    pltpu.sync_copy(scale_hbm, scalev)
    scale = scalev[...]
    lane = lax.broadcasted_iota(jnp.int32, (16,), 0)

    def group(g, carry):
        i0 = g * 16
        nd = jnp.minimum(i0 + lane, P - 1)
        pd = [plsc.load_gather(posv, [nd + c * P]) for c in range(3)]
        prd = [plsc.load_gather(projv, [nd + l * P]) for l in range(L)]
        acc = [jnp.zeros((16,), jnp.float32) for _ in range(LM)]
        iofs = nd * K
        for k in range(K):
            nidx = plsc.load_gather(idxv, [iofs + k])
            d2 = jnp.zeros((16,), jnp.float32)
            for c in range(3):
                ps = plsc.load_gather(posv, [nidx + c * P])
                df = ps - pd[c]
                d2 = d2 + df * df
            x = d2 + EPS
            xb = lax.bitcast_convert_type(x, jnp.int32)
            y = lax.bitcast_convert_type(
                (xb >> 1) + 0x1FBD1DF5, jnp.float32)
            for _ in range(3):
                y = 0.5 * (y + x / y)
            dist = y
            invd = 1.0 / (dist + EPS)
            t = dist * scale
            basis = [jnp.maximum(1.0 - jnp.abs(t - float(m)), 0.0)
                     for m in range(KS)]
            for l in range(L):
                pj = plsc.load_gather(projv, [nidx + l * P])
                dw = jnp.maximum((pj - prd[l]) * invd, 0.0)
                for m in range(KS):
                    acc[l * KS + m] = acc[l * KS + m] + dw * basis[m]
        ofs = (i0 + lane) * LM
        for lm in range(LM):
            plsc.store_scatter(abuf, [ofs + lm], acc[lm])
        return carry

    lax.fori_loop(0, NG, group, 0)
    pltpu.sync_copy(abuf, out_hbm.at[wid])


def _k3(a_ref, wf_ref, bdsc_ref, w1_ref, b1_ref, w2_ref, b2_ref,
        out_ref, ys_scr):
    b = pl.program_id(0)
    nb = pl.num_programs(0)
    a = a_ref[0]
    msg = jnp.dot(a, wf_ref[...], preferred_element_type=jnp.float32) / K
    msg = msg + bdsc_ref[...]
    y = 1.0 / (1.0 + jnp.exp(-msg))
    valid = lax.broadcasted_iota(jnp.int32, (PAD, FNR), 0) < P
    y = jnp.where(valid, y, 0.0)
    ys_scr[pl.ds(b, 1), :] = jnp.sum(y, axis=0, keepdims=True)

    @pl.when(b == nb - 1)
    def _():
        ys = ys_scr[...] * (1.0 / P)
        z = jnp.dot(ys, w1_ref[...],
                    preferred_element_type=jnp.float32) + b1_ref[...]
        h = jnp.where(z > 0, z, jnp.exp(jnp.minimum(z, 0.0)) - 1.0)
        logits = jnp.dot(h, w2_ref[...],
                         preferred_element_type=jnp.float32) + b2_ref[...]
        mx = jnp.max(logits, axis=1, keepdims=True)
        sh = logits - mx
        out_ref[...] = sh - jnp.log(jnp.sum(jnp.exp(sh), axis=1,
                                            keepdims=True))


def kernel(pos, edge_index, batch, dirs, W_spline, b_dsc, W1, b1, W2, b2):
    del edge_index, batch
    B = pos.shape[0] // P
    posB = pos.reshape(B, P, 3)
    pos3 = posB.transpose(0, 2, 1)
    wf = W_spline.reshape(LM, FNR)
    bdsc2 = b_dsc.reshape(1, FNR)

    idx, maxd2, proj = pl.pallas_call(
        _k1,
        grid=(B, PB),
        in_specs=[
            pl.BlockSpec((1, PBLK, 3), lambda b, rb: (b, rb, 0)),
            pl.BlockSpec((1, 3, P), lambda b, rb: (b, 0, 0)),
            pl.BlockSpec((L, 3), lambda b, rb: (0, 0)),
        ],
        out_specs=[
            pl.BlockSpec((1, PBLK, K), lambda b, rb: (b, rb, 0)),
            pl.BlockSpec((1, 1), lambda b, rb: (0, 0)),
            pl.BlockSpec((1, L, P), lambda b, rb: (b, 0, 0)),
        ],
        out_shape=[
            jax.ShapeDtypeStruct((B, P, K), jnp.int32),
            jax.ShapeDtypeStruct((1, 1), jnp.float32),
            jax.ShapeDtypeStruct((B, L, P), jnp.float32),
        ],
        interpret=_IP,
    )(posB, pos3, dirs)

    posp = pos3.reshape(B, 3 * P)
    projp = proj.reshape(B, L * P)
    idxp = idx.reshape(B, P * K)
    scale = (KS - 1) / (jnp.sqrt(maxd2[0, 0] + EPS) + EPS)
    scale16 = jnp.full((16,), 1.0, jnp.float32) * scale

    if _IP:
        a_t = _sc_emulate(posp, projp, idxp, scale16, B)
    else:
        mesh = plsc.VectorSubcoreMesh(core_axis_name="c", subcore_axis_name="s")
        sc = functools.partial(
            pl.kernel, _sc_body, mesh=mesh,
            compiler_params=pltpu.CompilerParams(needs_layout_passes=False),
            out_type=jax.ShapeDtypeStruct((B, PAD * LM), jnp.float32),
            scratch_types=[
                pltpu.VMEM((3 * P,), jnp.float32),
                pltpu.VMEM((L * P,), jnp.float32),
                pltpu.VMEM((P * K,), jnp.int32),
                pltpu.VMEM((16,), jnp.float32),
                pltpu.VMEM((PAD * LM,), jnp.float32),
            ],
        )()
        a_t = sc(posp, projp, idxp, scale16)
    a_t = a_t.reshape(B, PAD, LM)

    out = pl.pallas_call(
        _k3,
        grid=(B,),
        in_specs=[
            pl.BlockSpec((1, PAD, LM), lambda b: (b, 0, 0)),
            pl.BlockSpec((LM, FNR), lambda b: (0, 0)),
            pl.BlockSpec((1, FNR), lambda b: (0, 0)),
            pl.BlockSpec((FNR, 256), lambda b: (0, 0)),
            pl.BlockSpec((1, 256), lambda b: (0, 0)),
            pl.BlockSpec((256, W2.shape[1]), lambda b: (0, 0)),
            pl.BlockSpec((1, W2.shape[1]), lambda b: (0, 0)),
        ],
        out_specs=pl.BlockSpec((B, W2.shape[1]), lambda b: (0, 0)),
        out_shape=jax.ShapeDtypeStruct((B, W2.shape[1]), jnp.float32),
        scratch_shapes=[pltpu.VMEM((B, FNR), jnp.float32)],
        interpret=_IP,
    )(a_t, wf, bdsc2, W1, b1.reshape(1, -1), W2, b2.reshape(1, -1))
    return out


def _sc_emulate(posp, projp, idxp, scale16, B):
    pos = posp.reshape(B, 3, P)
    proj = projp.reshape(B, L, P)
    idx = idxp.reshape(B, P, K)
    scale = scale16[0]
    idxe = jnp.broadcast_to(idx.reshape(B, 1, P * K), (B, 3, P * K))
    g = jnp.take_along_axis(pos, idxe, axis=2).reshape(B, 3, P, K)
    d2 = jnp.sum((g - pos[:, :, :, None]) ** 2, axis=1)
    dist = jnp.sqrt(d2 + EPS)
    invd = 1.0 / (dist + EPS)
    t = dist * scale
    basis = jnp.maximum(1.0 - jnp.abs(t[..., None] -
                                      jnp.arange(KS, dtype=jnp.float32)), 0.0)
    idxe7 = jnp.broadcast_to(idx.reshape(B, 1, P * K), (B, L, P * K))
    pj = jnp.take_along_axis(proj, idxe7, axis=2).reshape(B, L, P, K)
    dw = jnp.maximum((pj - proj[:, :, :, None]) * invd[:, None], 0.0)
    a = jnp.einsum('blpk,bpkm->bplm', dw, basis).reshape(B, P, LM)
    a = jnp.pad(a, ((0, 0), (0, PAD - P), (0, 0)))
    return a.reshape(B, PAD * LM)


# speedup vs baseline: 1.1311x; 1.0393x over previous
import functools
import jax
import jax.numpy as jnp
from jax import lax
from jax.experimental import pallas as pl
from jax.experimental.pallas import tpu as pltpu
from jax.experimental.pallas import tpu_sc as plsc

P = 1000
K = 15
L = 7
KS = 5
FNR = 10
EPS = 1e-8
PBLK = 1000
PB = P // PBLK
BIG = 1e30
SENTF = 1.7014118346046923e38
LM = L * KS
PAD = 1008
NG = PAD // 16

_IP = False


def _k1(posPr_ref, pos3c_ref, dirs_ref, idx_ref, maxd2_ref, proj_ref):
    b = pl.program_id(0)
    rb = pl.program_id(1)
    pr = posPr_ref[0]
    pc = pos3c_ref[0]
    rn = jnp.sum(pr * pr, axis=1, keepdims=True)
    cn = jnp.sum(pc * pc, axis=0, keepdims=True)
    dotm = jnp.dot(pr, pc, preferred_element_type=jnp.float32)
    d2 = jnp.maximum(rn + cn - 2.0 * dotm, 0.0)
    riota = lax.broadcasted_iota(jnp.int32, (PBLK, P), 0) + rb * PBLK
    ciota = lax.broadcasted_iota(jnp.int32, (PBLK, P), 1)
    d2 = jnp.where(riota == ciota, BIG, d2)
    keys = (lax.bitcast_convert_type(d2, jnp.int32) & (-1024)) | ciota
    kf = lax.bitcast_convert_type(keys + 0x00800000, jnp.float32)
    mkf = jnp.min(kf, axis=1, keepdims=True)
    idx_cols = []
    for i in range(K):
        if i > 0:
            mkf = jnp.min(jnp.where(kf > mkf, kf, SENTF), axis=1,
                          keepdims=True)
        mki = lax.bitcast_convert_type(mkf, jnp.int32) - 0x00800000
        idx_cols.append(mki & 1023)
    idx_ref[0, :, :] = jnp.concatenate(idx_cols, axis=1)
    sel = kf <= mkf
    smax = jnp.max(jnp.where(sel, d2, -1.0), axis=(0, 1), keepdims=True)

    @pl.when((b == 0) & (rb == 0))
    def _():
        maxd2_ref[:, :] = jnp.full((1, 1), -1.0, jnp.float32)

    maxd2_ref[:, :] = jnp.maximum(maxd2_ref[:, :], smax)

    @pl.when(rb == 0)
    def _():
        dirs = dirs_ref[...]
        n = jnp.sqrt(jnp.sum(dirs * dirs, axis=1, keepdims=True))
        du = dirs / (n + EPS)
        proj_ref[0, :, :] = jnp.dot(du, pos3c_ref[0],
                                    preferred_element_type=jnp.float32)


def _sc_body(pos_hbm, proj_hbm, idx_hbm, scale_hbm, out_hbm,
             posv, projv, idxv, scalev, abuf):
    wid = lax.axis_index("s") * 2 + lax.axis_index("c")
    pltpu.sync_copy(pos_hbm.at[wid], posv)
    pltpu.sync_copy(proj_hbm.at[wid], projv)
    pltpu.sync_copy(idx_hbm.at[wid], idxv)
    pltpu.sync_copy(scale_hbm, scalev)
    scale = scalev[...]
    lane = lax.broadcasted_iota(jnp.int32, (16,), 0)

    def group(g, carry):
        i0 = g * 16
        nd = jnp.minimum(i0 + lane, P - 1)
        pd = [plsc.load_gather(posv, [nd + c * P]) for c in range(3)]
        prd = [plsc.load_gather(projv, [nd + l * P]) for l in range(L)]
        acc = [jnp.zeros((16,), jnp.float32) for _ in range(LM)]
        iofs = nd * K
        for k in range(K):
            nidx = plsc.load_gather(idxv, [iofs + k])
            d2 = jnp.zeros((16,), jnp.float32)
            for c in range(3):
                ps = plsc.load_gather(posv, [nidx + c * P])
                df = ps - pd[c]
                d2 = d2 + df * df
            x = d2 + EPS
            xb = lax.bitcast_convert_type(x, jnp.int32)
            y = lax.bitcast_convert_type(
                (xb >> 1) + 0x1FBD1DF5, jnp.float32)
            for _ in range(3):
                y = 0.5 * (y + x / y)
            dist = y
            invd = 1.0 / (dist + EPS)
            t = dist * scale
            basis = [jnp.maximum(1.0 - jnp.abs(t - float(m)), 0.0)
                     for m in range(KS)]
            for l in range(L):
                pj = plsc.load_gather(projv, [nidx + l * P])
                dw = jnp.maximum((pj - prd[l]) * invd, 0.0)
                for m in range(KS):
                    acc[l * KS + m] = acc[l * KS + m] + dw * basis[m]
        ofs = (i0 + lane) * LM
        for lm in range(LM):
            plsc.store_scatter(abuf, [ofs + lm], acc[lm])
        return carry

    lax.fori_loop(0, NG, group, 0)
    pltpu.sync_copy(abuf, out_hbm.at[wid])


def _k3(a_ref, wf_ref, bdsc_ref, w1_ref, b1_ref, w2_ref, b2_ref,
        out_ref, ys_scr):
    b = pl.program_id(0)
    nb = pl.num_programs(0)
    a = a_ref[0]
    msg = jnp.dot(a, wf_ref[...], preferred_element_type=jnp.float32) / K
    msg = msg + bdsc_ref[...]
    y = 1.0 / (1.0 + jnp.exp(-msg))
    valid = lax.broadcasted_iota(jnp.int32, (PAD, FNR), 0) < P
    y = jnp.where(valid, y, 0.0)
    ys_scr[pl.ds(b, 1), :] = jnp.sum(y, axis=0, keepdims=True)

    @pl.when(b == nb - 1)
    def _():
        ys = ys_scr[...] * (1.0 / P)
        z = jnp.dot(ys, w1_ref[...],
                    preferred_element_type=jnp.float32) + b1_ref[...]
        h = jnp.where(z > 0, z, jnp.exp(jnp.minimum(z, 0.0)) - 1.0)
        logits = jnp.dot(h, w2_ref[...],
                         preferred_element_type=jnp.float32) + b2_ref[...]
        mx = jnp.max(logits, axis=1, keepdims=True)
        sh = logits - mx
        out_ref[...] = sh - jnp.log(jnp.sum(jnp.exp(sh), axis=1,
                                            keepdims=True))


def kernel(pos, edge_index, batch, dirs, W_spline, b_dsc, W1, b1, W2, b2):
    del edge_index, batch
    B = pos.shape[0] // P
    posB = pos.reshape(B, P, 3)
    pos3 = posB.transpose(0, 2, 1)
    wf = W_spline.reshape(LM, FNR)
    bdsc2 = b_dsc.reshape(1, FNR)

    idx, maxd2, proj = pl.pallas_call(
        _k1,
        grid=(B, PB),
        in_specs=[
            pl.BlockSpec((1, PBLK, 3), lambda b, rb: (b, rb, 0)),
            pl.BlockSpec((1, 3, P), lambda b, rb: (b, 0, 0)),
            pl.BlockSpec((L, 3), lambda b, rb: (0, 0)),
        ],
        out_specs=[
            pl.BlockSpec((1, PBLK, K), lambda b, rb: (b, rb, 0)),
            pl.BlockSpec((1, 1), lambda b, rb: (0, 0)),
            pl.BlockSpec((1, L, P), lambda b, rb: (b, 0, 0)),
        ],
        out_shape=[
            jax.ShapeDtypeStruct((B, P, K), jnp.int32),
            jax.ShapeDtypeStruct((1, 1), jnp.float32),
            jax.ShapeDtypeStruct((B, L, P), jnp.float32),
        ],
        interpret=_IP,
    )(posB, pos3, dirs)

    posp = pos3.reshape(B, 3 * P)
    projp = proj.reshape(B, L * P)
    idxp = idx.reshape(B, P * K)
    scale = (KS - 1) / (jnp.sqrt(maxd2[0, 0] + EPS) + EPS)
    scale16 = jnp.full((16,), 1.0, jnp.float32) * scale

    if _IP:
        a_t = _sc_emulate(posp, projp, idxp, scale16, B)
    else:
        mesh = plsc.VectorSubcoreMesh(core_axis_name="c", subcore_axis_name="s")
        sc = functools.partial(
            pl.kernel, _sc_body, mesh=mesh,
            compiler_params=pltpu.CompilerParams(needs_layout_passes=False),
            out_type=jax.ShapeDtypeStruct((B, PAD * LM), jnp.float32),
            scratch_types=[
                pltpu.VMEM((3 * P,), jnp.float32),
                pltpu.VMEM((L * P,), jnp.float32),
                pltpu.VMEM((P * K,), jnp.int32),
                pltpu.VMEM((16,), jnp.float32),
                pltpu.VMEM((PAD * LM,), jnp.float32),
            ],
        )()
        a_t = sc(posp, projp, idxp, scale16)
    a_t = a_t.reshape(B, PAD, LM)

    out = pl.pallas_call(
        _k3,
        grid=(B,),
        in_specs=[
            pl.BlockSpec((1, PAD, LM), lambda b: (b, 0, 0)),
            pl.BlockSpec((LM, FNR), lambda b: (0, 0)),
            pl.BlockSpec((1, FNR), lambda b: (0, 0)),
            pl.BlockSpec((FNR, 256), lambda b: (0, 0)),
            pl.BlockSpec((1, 256), lambda b: (0, 0)),
            pl.BlockSpec((256, W2.shape[1]), lambda b: (0, 0)),
            pl.BlockSpec((1, W2.shape[1]), lambda b: (0, 0)),
        ],
        out_specs=pl.BlockSpec((B, W2.shape[1]), lambda b: (0, 0)),
        out_shape=jax.ShapeDtypeStruct((B, W2.shape[1]), jnp.float32),
        scratch_shapes=[pltpu.VMEM((B, FNR), jnp.float32)],
        interpret=_IP,
    )(a_t, wf, bdsc2, W1, b1.reshape(1, -1), W2, b2.reshape(1, -1))
    return out


def _sc_emulate(posp, projp, idxp, scale16, B):
    pos = posp.reshape(B, 3, P)
    proj = projp.reshape(B, L, P)
    idx = idxp.reshape(B, P, K)
    scale = scale16[0]
    idxe = jnp.broadcast_to(idx.reshape(B, 1, P * K), (B, 3, P * K))
    g = jnp.take_along_axis(pos, idxe, axis=2).reshape(B, 3, P, K)
    d2 = jnp.sum((g - pos[:, :, :, None]) ** 2, axis=1)
    dist = jnp.sqrt(d2 + EPS)
    invd = 1.0 / (dist + EPS)
    t = dist * scale
    basis = jnp.maximum(1.0 - jnp.abs(t[..., None] -
                                      jnp.arange(KS, dtype=jnp.float32)), 0.0)
    idxe7 = jnp.broadcast_to(idx.reshape(B, 1, P * K), (B, L, P * K))
    pj = jnp.take_along_axis(proj, idxe7, axis=2).reshape(B, L, P, K)
    dw = jnp.maximum((pj - proj[:, :, :, None]) * invd[:, None], 0.0)
    a = jnp.einsum('blpk,bpkm->bplm', dw, basis).reshape(B, P, LM)
    a = jnp.pad(a, ((0, 0), (0, PAD - P), (0, 0)))
    return a.reshape(B, PAD * LM)


# speedup vs baseline: 1.2148x; 1.0740x over previous
import functools
import jax
import jax.numpy as jnp
from jax import lax
from jax.experimental import pallas as pl
from jax.experimental.pallas import tpu as pltpu
from jax.experimental.pallas import tpu_sc as plsc

P = 1000
K = 15
L = 7
KS = 5
FNR = 10
EPS = 1e-8
PBLK = 1000
PB = P // PBLK
BIG = 1e30
SENTF = 1.7014118346046923e38
LM = L * KS
PAD = 1008
NG = PAD // 16

_IP = False


def _k1(posPr_ref, pos3c_ref, dirs_ref, idx_ref, maxd2_ref, proj_ref):
    b = pl.program_id(0)
    rb = pl.program_id(1)
    pr = posPr_ref[0]
    pc = pos3c_ref[0]
    rn = jnp.sum(pr * pr, axis=1, keepdims=True)
    cn = jnp.sum(pc * pc, axis=0, keepdims=True)
    dotm = jnp.dot(pr, pc, preferred_element_type=jnp.float32)
    d2 = jnp.maximum(rn + cn - 2.0 * dotm, 1e-35)
    riota = lax.broadcasted_iota(jnp.int32, (PBLK, P), 0) + rb * PBLK
    ciota = lax.broadcasted_iota(jnp.int32, (PBLK, P), 1)
    d2 = jnp.where(riota == ciota, BIG, d2)
    keys = (lax.bitcast_convert_type(d2, jnp.int32) & (-1024)) | ciota
    kf = lax.bitcast_convert_type(keys, jnp.float32)
    mkf = jnp.min(kf, axis=1, keepdims=True)
    idx_cols = []
    for i in range(K):
        if i > 0:
            mkf = jnp.min(jnp.where(kf > mkf, kf, SENTF), axis=1,
                          keepdims=True)
        mki = lax.bitcast_convert_type(mkf, jnp.int32)
        idx_cols.append(mki & 1023)
    idx_ref[0, :, :] = jnp.concatenate(idx_cols, axis=1)
    thd2 = lax.bitcast_convert_type(
        lax.bitcast_convert_type(mkf, jnp.int32) & (-1024), jnp.float32)
    smax = jnp.max(thd2, axis=(0, 1), keepdims=True)

    @pl.when((b == 0) & (rb == 0))
    def _():
        maxd2_ref[:, :] = jnp.full((1, 1), -1.0, jnp.float32)

    maxd2_ref[:, :] = jnp.maximum(maxd2_ref[:, :], smax)

    @pl.when(rb == 0)
    def _():
        dirs = dirs_ref[...]
        n = jnp.sqrt(jnp.sum(dirs * dirs, axis=1, keepdims=True))
        du = dirs / (n + EPS)
        proj_ref[0, :, :] = jnp.dot(du, pos3c_ref[0],
                                    preferred_element_type=jnp.float32)


def _sc_body(pos_hbm, proj_hbm, idx_hbm, scale_hbm, out_hbm,
             posv, projv, idxv, scalev, abuf):
    wid = lax.axis_index("s") * 2 + lax.axis_index("c")
    pltpu.sync_copy(pos_hbm.at[wid], posv)
    pltpu.sync_copy(proj_hbm.at[wid], projv)
    pltpu.sync_copy(idx_hbm.at[wid], idxv)
    pltpu.sync_copy(scale_hbm, scalev)
    scale = scalev[...]
    lane = lax.broadcasted_iota(jnp.int32, (16,), 0)

    def group(g, carry):
        i0 = g * 16
        nd = jnp.minimum(i0 + lane, P - 1)
        pd = [plsc.load_gather(posv, [nd + c * P]) for c in range(3)]
        prd = [plsc.load_gather(projv, [nd + l * P]) for l in range(L)]
        acc = [jnp.zeros((16,), jnp.float32) for _ in range(LM)]
        iofs = nd * K
        for k in range(K):
            nidx = plsc.load_gather(idxv, [iofs + k])
            d2 = jnp.zeros((16,), jnp.float32)
            for c in range(3):
                ps = plsc.load_gather(posv, [nidx + c * P])
                df = ps - pd[c]
                d2 = d2 + df * df
            x = d2 + EPS
            xb = lax.bitcast_convert_type(x, jnp.int32)
            y = lax.bitcast_convert_type(
                (xb >> 1) + 0x1FBD1DF5, jnp.float32)
            for _ in range(3):
                y = 0.5 * (y + x / y)
            dist = y
            invd = 1.0 / (dist + EPS)
            t = dist * scale
            basis = [jnp.maximum(1.0 - jnp.abs(t - float(m)), 0.0)
                     for m in range(KS)]
            for l in range(L):
                pj = plsc.load_gather(projv, [nidx + l * P])
                dw = jnp.maximum((pj - prd[l]) * invd, 0.0)
                for m in range(KS):
                    acc[l * KS + m] = acc[l * KS + m] + dw * basis[m]
        ofs = (i0 + lane) * LM
        for lm in range(LM):
            plsc.store_scatter(abuf, [ofs + lm], acc[lm])
        return carry

    lax.fori_loop(0, NG, group, 0)
    pltpu.sync_copy(abuf, out_hbm.at[wid])


def _k3(a_ref, wf_ref, bdsc_ref, w1_ref, b1_ref, w2_ref, b2_ref,
        out_ref, ys_scr):
    b = pl.program_id(0)
    nb = pl.num_programs(0)
    a = a_ref[0]
    msg = jnp.dot(a, wf_ref[...], preferred_element_type=jnp.float32) / K
    msg = msg + bdsc_ref[...]
    y = 1.0 / (1.0 + jnp.exp(-msg))
    valid = lax.broadcasted_iota(jnp.int32, (PAD, FNR), 0) < P
    y = jnp.where(valid, y, 0.0)
    ys_scr[pl.ds(b, 1), :] = jnp.sum(y, axis=0, keepdims=True)

    @pl.when(b == nb - 1)
    def _():
        ys = ys_scr[...] * (1.0 / P)
        z = jnp.dot(ys, w1_ref[...],
                    preferred_element_type=jnp.float32) + b1_ref[...]
        h = jnp.where(z > 0, z, jnp.exp(jnp.minimum(z, 0.0)) - 1.0)
        logits = jnp.dot(h, w2_ref[...],
                         preferred_element_type=jnp.float32) + b2_ref[...]
        mx = jnp.max(logits, axis=1, keepdims=True)
        sh = logits - mx
        out_ref[...] = sh - jnp.log(jnp.sum(jnp.exp(sh), axis=1,
                                            keepdims=True))


def kernel(pos, edge_index, batch, dirs, W_spline, b_dsc, W1, b1, W2, b2):
    del edge_index, batch
    B = pos.shape[0] // P
    posB = pos.reshape(B, P, 3)
    pos3 = posB.transpose(0, 2, 1)
    wf = W_spline.reshape(LM, FNR)
    bdsc2 = b_dsc.reshape(1, FNR)

    idx, maxd2, proj = pl.pallas_call(
        _k1,
        grid=(B, PB),
        in_specs=[
            pl.BlockSpec((1, PBLK, 3), lambda b, rb: (b, rb, 0)),
            pl.BlockSpec((1, 3, P), lambda b, rb: (b, 0, 0)),
            pl.BlockSpec((L, 3), lambda b, rb: (0, 0)),
        ],
        out_specs=[
            pl.BlockSpec((1, PBLK, K), lambda b, rb: (b, rb, 0)),
            pl.BlockSpec((1, 1), lambda b, rb: (0, 0)),
            pl.BlockSpec((1, L, P), lambda b, rb: (b, 0, 0)),
        ],
        out_shape=[
            jax.ShapeDtypeStruct((B, P, K), jnp.int32),
            jax.ShapeDtypeStruct((1, 1), jnp.float32),
            jax.ShapeDtypeStruct((B, L, P), jnp.float32),
        ],
        interpret=_IP,
    )(posB, pos3, dirs)

    posp = pos3.reshape(B, 3 * P)
    projp = proj.reshape(B, L * P)
    idxp = idx.reshape(B, P * K)
    scale = (KS - 1) / (jnp.sqrt(maxd2[0, 0] + EPS) + EPS)
    scale16 = jnp.full((16,), 1.0, jnp.float32) * scale

    if _IP:
        a_t = _sc_emulate(posp, projp, idxp, scale16, B)
    else:
        mesh = plsc.VectorSubcoreMesh(core_axis_name="c", subcore_axis_name="s")
        sc = functools.partial(
            pl.kernel, _sc_body, mesh=mesh,
            compiler_params=pltpu.CompilerParams(needs_layout_passes=False),
            out_type=jax.ShapeDtypeStruct((B, PAD * LM), jnp.float32),
            scratch_types=[
                pltpu.VMEM((3 * P,), jnp.float32),
                pltpu.VMEM((L * P,), jnp.float32),
                pltpu.VMEM((P * K,), jnp.int32),
                pltpu.VMEM((16,), jnp.float32),
                pltpu.VMEM((PAD * LM,), jnp.float32),
            ],
        )()
        a_t = sc(posp, projp, idxp, scale16)
    a_t = a_t.reshape(B, PAD, LM)

    out = pl.pallas_call(
        _k3,
        grid=(B,),
        in_specs=[
            pl.BlockSpec((1, PAD, LM), lambda b: (b, 0, 0)),
            pl.BlockSpec((LM, FNR), lambda b: (0, 0)),
            pl.BlockSpec((1, FNR), lambda b: (0, 0)),
            pl.BlockSpec((FNR, 256), lambda b: (0, 0)),
            pl.BlockSpec((1, 256), lambda b: (0, 0)),
            pl.BlockSpec((256, W2.shape[1]), lambda b: (0, 0)),
            pl.BlockSpec((1, W2.shape[1]), lambda b: (0, 0)),
        ],
        out_specs=pl.BlockSpec((B, W2.shape[1]), lambda b: (0, 0)),
        out_shape=jax.ShapeDtypeStruct((B, W2.shape[1]), jnp.float32),
        scratch_shapes=[pltpu.VMEM((B, FNR), jnp.float32)],
        interpret=_IP,
    )(a_t, wf, bdsc2, W1, b1.reshape(1, -1), W2, b2.reshape(1, -1))
    return out


def _sc_emulate(posp, projp, idxp, scale16, B):
    pos = posp.reshape(B, 3, P)
    proj = projp.reshape(B, L, P)
    idx = idxp.reshape(B, P, K)
    scale = scale16[0]
    idxe = jnp.broadcast_to(idx.reshape(B, 1, P * K), (B, 3, P * K))
    g = jnp.take_along_axis(pos, idxe, axis=2).reshape(B, 3, P, K)
    d2 = jnp.sum((g - pos[:, :, :, None]) ** 2, axis=1)
    dist = jnp.sqrt(d2 + EPS)
    invd = 1.0 / (dist + EPS)
    t = dist * scale
    basis = jnp.maximum(1.0 - jnp.abs(t[..., None] -
                                      jnp.arange(KS, dtype=jnp.float32)), 0.0)
    idxe7 = jnp.broadcast_to(idx.reshape(B, 1, P * K), (B, L, P * K))
    pj = jnp.take_along_axis(proj, idxe7, axis=2).reshape(B, L, P, K)
    dw = jnp.maximum((pj - proj[:, :, :, None]) * invd[:, None], 0.0)
    a = jnp.einsum('blpk,bpkm->bplm', dw, basis).reshape(B, P, LM)
    a = jnp.pad(a, ((0, 0), (0, PAD - P), (0, 0)))
    return a.reshape(B, PAD * LM)


# speedup vs baseline: 1.2148x; 1.0000x over previous
import functools
import jax
import jax.numpy as jnp
from jax import lax
from jax.experimental import pallas as pl
from jax.experimental.pallas import tpu as pltpu
from jax.experimental.pallas import tpu_sc as plsc

P = 1000
K = 15
L = 7
KS = 5
FNR = 10
EPS = 1e-8
PBLK = 1000
PB = P // PBLK
BIG = 1e30
SENTF = 1.7014118346046923e38
LM = L * KS
PAD = 1008
NG = PAD // 16


def _k1(posPr_ref, pos3c_ref, dirs_ref, idx_ref, maxd2_ref, proj_ref):
    b = pl.program_id(0)
    rb = pl.program_id(1)
    pr = posPr_ref[0]
    pc = pos3c_ref[0]
    rn = jnp.sum(pr * pr, axis=1, keepdims=True)
    cn = jnp.sum(pc * pc, axis=0, keepdims=True)
    dotm = jnp.dot(pr, pc, preferred_element_type=jnp.float32)
    d2 = jnp.maximum(rn + cn - 2.0 * dotm, 1e-35)
    riota = lax.broadcasted_iota(jnp.int32, (PBLK, P), 0) + rb * PBLK
    ciota = lax.broadcasted_iota(jnp.int32, (PBLK, P), 1)
    d2 = jnp.where(riota == ciota, BIG, d2)
    keys = (lax.bitcast_convert_type(d2, jnp.int32) & (-1024)) | ciota
    kf = lax.bitcast_convert_type(keys, jnp.float32)
    mkf = jnp.min(kf, axis=1, keepdims=True)
    idx_cols = []
    for i in range(K):
        if i > 0:
            mkf = jnp.min(jnp.where(kf > mkf, kf, SENTF), axis=1,
                          keepdims=True)
        mki = lax.bitcast_convert_type(mkf, jnp.int32)
        idx_cols.append(mki & 1023)
    idx_ref[0, :, :] = jnp.concatenate(idx_cols, axis=1)
    thd2 = lax.bitcast_convert_type(
        lax.bitcast_convert_type(mkf, jnp.int32) & (-1024), jnp.float32)
    smax = jnp.max(thd2, axis=(0, 1), keepdims=True)

    @pl.when((b == 0) & (rb == 0))
    def _():
        maxd2_ref[:, :] = jnp.full((1, 1), -1.0, jnp.float32)

    maxd2_ref[:, :] = jnp.maximum(maxd2_ref[:, :], smax)

    @pl.when(rb == 0)
    def _():
        dirs = dirs_ref[...]
        n = jnp.sqrt(jnp.sum(dirs * dirs, axis=1, keepdims=True))
        du = dirs / (n + EPS)
        proj_ref[0, :, :] = jnp.dot(du, pos3c_ref[0],
                                    preferred_element_type=jnp.float32)


def _sc_body(pos_hbm, proj_hbm, idx_hbm, scale_hbm, out_hbm,
             posv, projv, idxv, scalev, abuf):
    wid = lax.axis_index("s") * 2 + lax.axis_index("c")
    pltpu.sync_copy(pos_hbm.at[wid], posv)
    pltpu.sync_copy(proj_hbm.at[wid], projv)
    pltpu.sync_copy(idx_hbm.at[wid], idxv)
    pltpu.sync_copy(scale_hbm, scalev)
    scale = scalev[...]
    lane = lax.broadcasted_iota(jnp.int32, (16,), 0)

    def group(g, carry):
        i0 = g * 16
        nd = jnp.minimum(i0 + lane, P - 1)
        pd = [plsc.load_gather(posv, [nd + c * P]) for c in range(3)]
        prd = [plsc.load_gather(projv, [nd + l * P]) for l in range(L)]
        acc = [jnp.zeros((16,), jnp.float32) for _ in range(LM)]
        iofs = nd * K
        for k in range(K):
            nidx = plsc.load_gather(idxv, [iofs + k])
            d2 = jnp.zeros((16,), jnp.float32)
            for c in range(3):
                ps = plsc.load_gather(posv, [nidx + c * P])
                df = ps - pd[c]
                d2 = d2 + df * df
            x = d2 + EPS
            xb = lax.bitcast_convert_type(x, jnp.int32)
            y = lax.bitcast_convert_type(
                (xb >> 1) + 0x1FBD1DF5, jnp.float32)
            for _ in range(3):
                y = 0.5 * (y + x / y)
            dist = y
            invd = 1.0 / (dist + EPS)
            t = dist * scale
            basis = [jnp.maximum(1.0 - jnp.abs(t - float(m)), 0.0)
                     for m in range(KS)]
            for l in range(L):
                pj = plsc.load_gather(projv, [nidx + l * P])
                dw = jnp.maximum((pj - prd[l]) * invd, 0.0)
                for m in range(KS):
                    acc[l * KS + m] = acc[l * KS + m] + dw * basis[m]
        ofs = (i0 + lane) * LM
        for lm in range(LM):
            plsc.store_scatter(abuf, [ofs + lm], acc[lm])
        return carry

    lax.fori_loop(0, NG, group, 0)
    pltpu.sync_copy(abuf, out_hbm.at[wid])


def _k3(a_ref, wf_ref, bdsc_ref, w1_ref, b1_ref, w2_ref, b2_ref,
        out_ref, ys_scr):
    b = pl.program_id(0)
    nb = pl.num_programs(0)
    a = a_ref[0]
    msg = jnp.dot(a, wf_ref[...], preferred_element_type=jnp.float32) / K
    msg = msg + bdsc_ref[...]
    y = 1.0 / (1.0 + jnp.exp(-msg))
    valid = lax.broadcasted_iota(jnp.int32, (PAD, FNR), 0) < P
    y = jnp.where(valid, y, 0.0)
    ys_scr[pl.ds(b, 1), :] = jnp.sum(y, axis=0, keepdims=True)

    @pl.when(b == nb - 1)
    def _():
        ys = ys_scr[...] * (1.0 / P)
        z = jnp.dot(ys, w1_ref[...],
                    preferred_element_type=jnp.float32) + b1_ref[...]
        h = jnp.where(z > 0, z, jnp.exp(jnp.minimum(z, 0.0)) - 1.0)
        logits = jnp.dot(h, w2_ref[...],
                         preferred_element_type=jnp.float32) + b2_ref[...]
        mx = jnp.max(logits, axis=1, keepdims=True)
        sh = logits - mx
        out_ref[...] = sh - jnp.log(jnp.sum(jnp.exp(sh), axis=1,
                                            keepdims=True))


def kernel(pos, edge_index, batch, dirs, W_spline, b_dsc, W1, b1, W2, b2):
    del edge_index, batch
    B = pos.shape[0] // P
    posB = pos.reshape(B, P, 3)
    pos3 = posB.transpose(0, 2, 1)
    wf = W_spline.reshape(LM, FNR)
    bdsc2 = b_dsc.reshape(1, FNR)

    idx, maxd2, proj = pl.pallas_call(
        _k1,
        grid=(B, PB),
        in_specs=[
            pl.BlockSpec((1, PBLK, 3), lambda b, rb: (b, rb, 0)),
            pl.BlockSpec((1, 3, P), lambda b, rb: (b, 0, 0)),
            pl.BlockSpec((L, 3), lambda b, rb: (0, 0)),
        ],
        out_specs=[
            pl.BlockSpec((1, PBLK, K), lambda b, rb: (b, rb, 0)),
            pl.BlockSpec((1, 1), lambda b, rb: (0, 0)),
            pl.BlockSpec((1, L, P), lambda b, rb: (b, 0, 0)),
        ],
        out_shape=[
            jax.ShapeDtypeStruct((B, P, K), jnp.int32),
            jax.ShapeDtypeStruct((1, 1), jnp.float32),
            jax.ShapeDtypeStruct((B, L, P), jnp.float32),
        ],
    )(posB, pos3, dirs)

    posp = pos3.reshape(B, 3 * P)
    projp = proj.reshape(B, L * P)
    idxp = idx.reshape(B, P * K)
    scale = (KS - 1) / (jnp.sqrt(maxd2[0, 0] + EPS) + EPS)
    scale16 = jnp.full((16,), 1.0, jnp.float32) * scale

    mesh = plsc.VectorSubcoreMesh(core_axis_name="c", subcore_axis_name="s")
    sc = functools.partial(
        pl.kernel, _sc_body, mesh=mesh,
        compiler_params=pltpu.CompilerParams(needs_layout_passes=False),
        out_type=jax.ShapeDtypeStruct((B, PAD * LM), jnp.float32),
        scratch_types=[
            pltpu.VMEM((3 * P,), jnp.float32),
            pltpu.VMEM((L * P,), jnp.float32),
            pltpu.VMEM((P * K,), jnp.int32),
            pltpu.VMEM((16,), jnp.float32),
            pltpu.VMEM((PAD * LM,), jnp.float32),
        ],
    )()
    a_t = sc(posp, projp, idxp, scale16).reshape(B, PAD, LM)

    out = pl.pallas_call(
        _k3,
        grid=(B,),
        in_specs=[
            pl.BlockSpec((1, PAD, LM), lambda b: (b, 0, 0)),
            pl.BlockSpec((LM, FNR), lambda b: (0, 0)),
            pl.BlockSpec((1, FNR), lambda b: (0, 0)),
            pl.BlockSpec((FNR, 256), lambda b: (0, 0)),
            pl.BlockSpec((1, 256), lambda b: (0, 0)),
            pl.BlockSpec((256, W2.shape[1]), lambda b: (0, 0)),
            pl.BlockSpec((1, W2.shape[1]), lambda b: (0, 0)),
        ],
        out_specs=pl.BlockSpec((B, W2.shape[1]), lambda b: (0, 0)),
        out_shape=jax.ShapeDtypeStruct((B, W2.shape[1]), jnp.float32),
        scratch_shapes=[pltpu.VMEM((B, FNR), jnp.float32)],
    )(a_t, wf, bdsc2, W1, b1.reshape(1, -1), W2, b2.reshape(1, -1))
    return out
